# Initial kernel scaffold; baseline (speedup 1.0000x reference)
import jax
import jax.numpy as jnp
from jax.experimental import pallas as pl


def kernel(x, edge_index, W1, b1, W2, b2):
    raise NotImplementedError("write your pallas kernel here")


# speedup vs baseline: 13.1383x; 13.1383x over previous
import functools

import jax
import jax.numpy as jnp
from jax import lax
from jax.experimental import pallas as pl
from jax.experimental.pallas import tpu as pltpu
from jax.experimental.pallas import tpu_sc as plsc

N = 10000
E = 320000
D = 128
C = 8
NCORES = 2
NTILES = 16
CHUNK = 128
STEPS = 80
EP = NCORES * NTILES * STEPS * CHUNK
NP = 10240
RPT = NP // NTILES
BLK = 256


def _sc_mesh():
    return plsc.VectorSubcoreMesh(core_axis_name="c", subcore_axis_name="s")


def _sc_degree(dst2d):
    @functools.partial(
        pl.kernel,
        out_type=jax.ShapeDtypeStruct((NCORES, NP), jnp.float32),
        mesh=_sc_mesh(),
        compiler_params=pltpu.CompilerParams(use_tc_tiling_on_sc=False),
        scratch_types=[
            pltpu.VMEM((STEPS, CHUNK), jnp.int32),
            pltpu.VMEM((CHUNK,), jnp.float32),
            pltpu.VMEM((RPT,), jnp.float32),
            pltpu.VMEM_SHARED((NP,), jnp.float32),
        ],
    )
    def k(dst_hbm, out_hbm, idx_v, ones_v, zb_v, acc):
        cid = lax.axis_index("c")
        sid = lax.axis_index("s")
        wid = cid * NTILES + sid
        one16 = jnp.ones((16,), jnp.float32)
        zero16 = jnp.zeros((16,), jnp.float32)
        for i in range(CHUNK // 16):
            ones_v[pl.ds(i * 16, 16)] = one16
        for i in range(RPT // 16):
            zb_v[pl.ds(i * 16, 16)] = zero16
        pltpu.sync_copy(zb_v, acc.at[pl.ds(sid * RPT, RPT)])
        pltpu.sync_copy(dst_hbm.at[pl.ds(wid * STEPS, STEPS)], idx_v)
        plsc.subcore_barrier()

        def body(j, carry):
            pltpu.sync_copy(ones_v, acc.at[idx_v.at[j]], add=True)
            return carry

        lax.fori_loop(0, STEPS, body, 0)
        plsc.subcore_barrier()
        pltpu.sync_copy(acc.at[pl.ds(sid * RPT, RPT)],
                        out_hbm.at[cid, pl.ds(sid * RPT, RPT)])

    return k(dst2d)


def _sc_aggregate(hs, src2d, dst2d, feat):
    npairs = STEPS // 2

    @functools.partial(
        pl.kernel,
        out_type=jax.ShapeDtypeStruct((NCORES, NP, feat), jnp.float32),
        mesh=_sc_mesh(),
        compiler_params=pltpu.CompilerParams(use_tc_tiling_on_sc=False),
        scratch_types=[
            pltpu.VMEM((STEPS, CHUNK), jnp.int32),
            pltpu.VMEM((STEPS, CHUNK), jnp.int32),
            pltpu.VMEM((CHUNK, feat), jnp.float32),
            pltpu.VMEM((CHUNK, feat), jnp.float32),
            pltpu.VMEM_SHARED((NP, feat), jnp.float32),
            pltpu.SemaphoreType.DMA,
            pltpu.SemaphoreType.DMA,
        ],
    )
    def k(hs_hbm, src_hbm, dst_hbm, out_hbm,
          src_v, dst_v, buf0, buf1, acc, sem0, sem1):
        cid = lax.axis_index("c")
        sid = lax.axis_index("s")
        wid = cid * NTILES + sid
        rbase = sid * RPT
        pltpu.sync_copy(hs_hbm.at[pl.ds(NP - CHUNK, CHUNK)], buf0)
        for r in range(RPT // CHUNK):
            pltpu.sync_copy(buf0, acc.at[pl.ds(rbase + r * CHUNK, CHUNK)])
        pltpu.sync_copy(src_hbm.at[pl.ds(wid * STEPS, STEPS)], src_v)
        pltpu.sync_copy(dst_hbm.at[pl.ds(wid * STEPS, STEPS)], dst_v)
        plsc.subcore_barrier()

        pltpu.async_copy(hs_hbm.at[src_v.at[0]], buf0, sem0)

        def body(jj, carry):
            j0 = jj * 2
            pltpu.async_copy(hs_hbm.at[src_v.at[j0 + 1]], buf1, sem1)
            pltpu.make_async_copy(hs_hbm.at[src_v.at[j0]], buf0, sem0).wait()
            pltpu.sync_copy(buf0, acc.at[dst_v.at[j0]], add=True)
            pltpu.async_copy(hs_hbm.at[src_v.at[j0 + 2]], buf0, sem0)
            pltpu.make_async_copy(hs_hbm.at[src_v.at[j0 + 1]], buf1, sem1).wait()
            pltpu.sync_copy(buf1, acc.at[dst_v.at[j0 + 1]], add=True)
            return carry

        lax.fori_loop(0, npairs - 1, body, 0)
        j0 = STEPS - 2
        pltpu.async_copy(hs_hbm.at[src_v.at[j0 + 1]], buf1, sem1)
        pltpu.make_async_copy(hs_hbm.at[src_v.at[j0]], buf0, sem0).wait()
        pltpu.sync_copy(buf0, acc.at[dst_v.at[j0]], add=True)
        pltpu.make_async_copy(hs_hbm.at[src_v.at[j0 + 1]], buf1, sem1).wait()
        pltpu.sync_copy(buf1, acc.at[dst_v.at[j0 + 1]], add=True)

        plsc.subcore_barrier()
        pltpu.sync_copy(acc.at[pl.ds(rbase, RPT)],
                        out_hbm.at[cid, pl.ds(rbase, RPT)])

    return k(hs, src2d, dst2d)


DH = D // 2


def _tc_scale_matmul(x_p, W1, dpt):
    def body(x_ref, w_ref, dp_ref, hsa_ref, hsb_ref, di_ref):
        i = pl.program_id(0)
        deg = dp_ref[:, 0:1] + dp_ref[:, 1:2] + 1.0
        rows = lax.broadcasted_iota(jnp.int32, (BLK, 1), 0) + i * BLK
        dinv = jnp.where(rows < N, lax.rsqrt(deg), 0.0)
        h = jnp.dot(x_ref[...], w_ref[...], preferred_element_type=jnp.float32)
        hs = h * dinv
        hsa_ref[...] = hs[:, :DH]
        hsb_ref[...] = hs[:, DH:]
        di_ref[...] = dinv

    return pl.pallas_call(
        body,
        grid=(NP // BLK,),
        in_specs=[
            pl.BlockSpec((BLK, D), lambda i: (i, 0)),
            pl.BlockSpec((D, D), lambda i: (0, 0)),
            pl.BlockSpec((BLK, 2), lambda i: (i, 0)),
        ],
        out_specs=[
            pl.BlockSpec((BLK, DH), lambda i: (i, 0)),
            pl.BlockSpec((BLK, DH), lambda i: (i, 0)),
            pl.BlockSpec((BLK, 1), lambda i: (i, 0)),
        ],
        out_shape=[
            jax.ShapeDtypeStruct((NP, DH), jnp.float32),
            jax.ShapeDtypeStruct((NP, DH), jnp.float32),
            jax.ShapeDtypeStruct((NP, 1), jnp.float32),
        ],
    )(x_p, W1, dpt)


def _tc_layer2_input(pa0, pa1, pb0, pb1, h1sa, h1sb, dinv, b1a, b1b, W2a, W2b):
    def body(pa0_ref, pa1_ref, pb0_ref, pb1_ref, ha_ref, hb_ref, di_ref,
             ba_ref, bb_ref, wa_ref, wb_ref, o_ref):
        di = di_ref[...]
        al = di * (pa0_ref[...] + pa1_ref[...] + ha_ref[...]) + ba_ref[...]
        ar = di * (pb0_ref[...] + pb1_ref[...] + hb_ref[...]) + bb_ref[...]
        al = jnp.maximum(al, 0.0)
        ar = jnp.maximum(ar, 0.0)
        h2 = (jnp.dot(al, wa_ref[...], preferred_element_type=jnp.float32)
              + jnp.dot(ar, wb_ref[...], preferred_element_type=jnp.float32))
        o_ref[...] = h2 * di

    half = pl.BlockSpec((BLK, DH), lambda i: (i, 0))
    return pl.pallas_call(
        body,
        grid=(NP // BLK,),
        in_specs=[
            half, half, half, half, half, half,
            pl.BlockSpec((BLK, 1), lambda i: (i, 0)),
            pl.BlockSpec((1, DH), lambda i: (0, 0)),
            pl.BlockSpec((1, DH), lambda i: (0, 0)),
            pl.BlockSpec((DH, C), lambda i: (0, 0)),
            pl.BlockSpec((DH, C), lambda i: (0, 0)),
        ],
        out_specs=pl.BlockSpec((BLK, C), lambda i: (i, 0)),
        out_shape=jax.ShapeDtypeStruct((NP, C), jnp.float32),
    )(pa0, pa1, pb0, pb1, h1sa, h1sb, dinv, b1a, b1b, W2a, W2b)


def _tc_final(pa, pb, h2s, dinv, b2):
    def body(pa_ref, pb_ref, h_ref, di_ref, b_ref, o_ref):
        o_ref[...] = (di_ref[...] * (pa_ref[...] + pb_ref[...] + h_ref[...])
                      + b_ref[...])

    return pl.pallas_call(
        body,
        grid=(NP // BLK,),
        in_specs=[
            pl.BlockSpec((BLK, C), lambda i: (i, 0)),
            pl.BlockSpec((BLK, C), lambda i: (i, 0)),
            pl.BlockSpec((BLK, C), lambda i: (i, 0)),
            pl.BlockSpec((BLK, 1), lambda i: (i, 0)),
            pl.BlockSpec((1, C), lambda i: (0, 0)),
        ],
        out_specs=pl.BlockSpec((BLK, C), lambda i: (i, 0)),
        out_shape=jax.ShapeDtypeStruct((NP, C), jnp.float32),
    )(pa, pb, h2s, dinv, b2)


@jax.jit
def kernel(x, edge_index, W1, b1, W2, b2):
    pad_e = EP - E
    src = jnp.concatenate(
        [edge_index[0], jnp.full((pad_e,), N, jnp.int32)]).reshape(-1, CHUNK)
    dst = jnp.concatenate(
        [edge_index[1], jnp.full((pad_e,), N, jnp.int32)]).reshape(-1, CHUNK)
    x_p = jnp.concatenate([x, jnp.zeros((NP - N, D), x.dtype)])

    dp = _sc_degree(dst)
    h1sa, h1sb, dinv = _tc_scale_matmul(x_p, W1, dp.T)
    p1a = _sc_aggregate(h1sa, src, dst, DH)
    p1b = _sc_aggregate(h1sb, src, dst, DH)
    h2s = _tc_layer2_input(p1a[0], p1a[1], p1b[0], p1b[1], h1sa, h1sb, dinv,
                           b1[:DH].reshape(1, DH), b1[DH:].reshape(1, DH),
                           W2[:DH], W2[DH:])
    p2 = _sc_aggregate(h2s, src, dst, C)
    out = _tc_final(p2[0], p2[1], h2s, dinv, b2.reshape(1, C))
    return out[:N]
